# baseline (speedup 1.0000x reference)
import functools

import jax
import jax.numpy as jnp
import numpy as np
from jax import lax
from jax.experimental import pallas as pl
from jax.experimental.pallas import tpu as pltpu
from jax.experimental.pallas import tpu_sc as plsc

_T = 512
_VOCAB = 100000
_NSUB = 16
_CHUNK = 6272
_NSCAN = _CHUNK // 16
_HALF = _VOCAB // 2
_OUT_CHUNK = 3136
_NOUT = _OUT_CHUNK // 16


def _rotl(x, r):
    return ((x << np.uint32(r)) | (x >> np.uint32(32 - r))).astype(np.uint32)


def _threefry2x32(k0, k1, x0, x1):
    rot = ((13, 15, 26, 6), (17, 29, 16, 24))
    ks = (np.uint32(k0), np.uint32(k1),
          np.uint32(k0 ^ k1 ^ np.uint32(0x1BD11BDA)))
    x0 = (x0 + ks[0]).astype(np.uint32)
    x1 = (x1 + ks[1]).astype(np.uint32)
    for i in range(5):
        for r in rot[i % 2]:
            x0 = (x0 + x1).astype(np.uint32)
            x1 = _rotl(x1, r)
            x1 = x1 ^ x0
        x0 = (x0 + ks[(i + 1) % 3]).astype(np.uint32)
        x1 = (x1 + ks[(i + 2) % 3] + np.uint32(i + 1)).astype(np.uint32)
    return x0, x1


def _gumbel_f32(seed, n):
    k0 = np.uint32(np.uint64(seed) >> np.uint64(32))
    k1 = np.uint32(np.uint64(seed) & np.uint64(0xFFFFFFFF))
    count = np.arange(n, dtype=np.uint32)
    o0, o1 = _threefry2x32(k0, k1, np.zeros(n, np.uint32), count)
    bits = o0 ^ o1
    float_bits = (bits >> np.uint32(9)) | np.uint32(0x3F800000)
    floats = float_bits.view(np.float32) - np.float32(1.0)
    tiny = np.float32(np.finfo(np.float32).tiny)
    u = np.maximum(tiny, floats * (np.float32(1.0) - tiny) + tiny)
    return -np.log(-np.log(u.astype(np.float32))).astype(np.float32)


_EXP_GUMBEL = np.exp(_gumbel_f32(42, _VOCAB).astype(np.float64)).astype(
    np.float32)

_GDN = jax.lax.GatherDimensionNumbers(
    offset_dims=(), collapsed_slice_dims=(0,), start_index_map=(0,))


def _take16(v, perm):
    return lax.gather(
        v, perm[:, None], _GDN, slice_sizes=(1,),
        mode=lax.GatherScatterMode.PROMISE_IN_BOUNDS)


def _lex_better(v, bv, gi, bi):
    return (v > bv) | ((v == bv) & (gi < bi))


def _sc_body(t_hbm, w_hbm, eg_hbm, out_hbm,
             tvec_v, rows_v, eg_v, bv_v, bi_v, shv, shi, allv_v, alli_v,
             outbuf_v, sem):
    cid = lax.axis_index("c")
    sid = lax.axis_index("s")
    lanes = lax.iota(jnp.int32, 16)

    start = jnp.minimum(sid * _CHUNK, _VOCAB - _CHUNK)
    pltpu.sync_copy(t_hbm, tvec_v)
    t = lax.squeeze(lax.slice(tvec_v[...], (0,), (1,)), (0,))
    gather = pltpu.async_copy(
        w_hbm.at[pl.ds(t * _VOCAB + start, _CHUNK)], rows_v, sem)
    pltpu.sync_copy(eg_hbm.at[pl.ds(start, _CHUNK)], eg_v)
    gather.wait()

    def scan_body(i, carry):
        bv, bi = carry
        off = i * 16
        v = rows_v[pl.ds(off, 16)] * eg_v[pl.ds(off, 16)]
        gi = start + off + lanes
        better = _lex_better(v, bv, gi, bi)
        return jnp.where(better, v, bv), jnp.where(better, gi, bi)

    bv, bi = lax.fori_loop(
        0, _NSCAN, scan_body,
        (jnp.full((16,), -jnp.inf, jnp.float32),
         jnp.zeros((16,), jnp.int32)))

    bv_v[...] = bv
    bi_v[...] = bi
    pltpu.sync_copy(bv_v, shv.at[pl.ds(sid * 16, 16)])
    pltpu.sync_copy(bi_v, shi.at[pl.ds(sid * 16, 16)])
    plsc.subcore_barrier()
    pltpu.sync_copy(shv, allv_v)
    pltpu.sync_copy(shi, alli_v)
    gv = allv_v[pl.ds(0, 16)]
    gi = alli_v[pl.ds(0, 16)]
    for j in range(1, _NSUB):
        v = allv_v[pl.ds(j * 16, 16)]
        ii = alli_v[pl.ds(j * 16, 16)]
        better = _lex_better(v, gv, ii, gi)
        gv = jnp.where(better, v, gv)
        gi = jnp.where(better, ii, gi)

    for s in (8, 4, 2, 1):
        perm = lanes ^ s
        ov = _take16(gv, perm)
        oi = _take16(gi, perm)
        better = _lex_better(ov, gv, oi, gi)
        gv = jnp.where(better, ov, gv)
        gi = jnp.where(better, oi, gi)
    amax = gi

    out_start = cid * _HALF + jnp.minimum(sid * _OUT_CHUNK, _HALF - _OUT_CHUNK)

    def fill_body(j, carry):
        off = j * 16
        gidx = out_start + off + lanes
        outbuf_v[pl.ds(off, 16)] = jnp.where(gidx == amax, 0.0, 1000.0)
        return carry

    lax.fori_loop(0, _NOUT, fill_body, 0)
    pltpu.sync_copy(outbuf_v, out_hbm.at[pl.ds(out_start, _OUT_CHUNK)])


@functools.cache
def _sc_call():
    return pl.kernel(
        _sc_body,
        out_type=jax.ShapeDtypeStruct((_VOCAB,), jnp.float32),
        mesh=plsc.VectorSubcoreMesh(core_axis_name="c", subcore_axis_name="s"),
        scratch_types=[
            pltpu.VMEM((16,), jnp.int32),
            pltpu.VMEM((_CHUNK,), jnp.float32),
            pltpu.VMEM((_CHUNK,), jnp.float32),
            pltpu.VMEM((16,), jnp.float32),
            pltpu.VMEM((16,), jnp.int32),
            pltpu.VMEM_SHARED((256,), jnp.float32),
            pltpu.VMEM_SHARED((256,), jnp.int32),
            pltpu.VMEM((256,), jnp.float32),
            pltpu.VMEM((256,), jnp.int32),
            pltpu.VMEM((_OUT_CHUNK,), jnp.float32),
            pltpu.SemaphoreType.DMA,
        ],
    )


def kernel(weights, time):
    eg = jnp.asarray(_EXP_GUMBEL)
    t16 = jnp.full((16,), jnp.asarray(time, jnp.int32))
    return _sc_call()(t16, weights.reshape(-1), eg)


# speedup vs baseline: 2.2304x; 2.2304x over previous
import functools

import jax
import jax.numpy as jnp
import numpy as np
from jax import lax
from jax.experimental import pallas as pl
from jax.experimental.pallas import tpu as pltpu
from jax.experimental.pallas import tpu_sc as plsc

_T = 512
_VOCAB = 100000
_NSUB = 16
_CHUNK = 6272
_NSCAN = _CHUNK // 16
_HALF = _VOCAB // 2
_TAIL = 99968
_OUT_CHUNK = 3136
_NOUT = _OUT_CHUNK // 16


def _rotl(x, r):
    return ((x << np.uint32(r)) | (x >> np.uint32(32 - r))).astype(np.uint32)


def _threefry2x32(k0, k1, x0, x1):
    rot = ((13, 15, 26, 6), (17, 29, 16, 24))
    ks = (np.uint32(k0), np.uint32(k1),
          np.uint32(k0 ^ k1 ^ np.uint32(0x1BD11BDA)))
    x0 = (x0 + ks[0]).astype(np.uint32)
    x1 = (x1 + ks[1]).astype(np.uint32)
    for i in range(5):
        for r in rot[i % 2]:
            x0 = (x0 + x1).astype(np.uint32)
            x1 = _rotl(x1, r)
            x1 = x1 ^ x0
        x0 = (x0 + ks[(i + 1) % 3]).astype(np.uint32)
        x1 = (x1 + ks[(i + 2) % 3] + np.uint32(i + 1)).astype(np.uint32)
    return x0, x1


def _gumbel_f32(seed, n):
    k0 = np.uint32(np.uint64(seed) >> np.uint64(32))
    k1 = np.uint32(np.uint64(seed) & np.uint64(0xFFFFFFFF))
    count = np.arange(n, dtype=np.uint32)
    o0, o1 = _threefry2x32(k0, k1, np.zeros(n, np.uint32), count)
    bits = o0 ^ o1
    float_bits = (bits >> np.uint32(9)) | np.uint32(0x3F800000)
    floats = float_bits.view(np.float32) - np.float32(1.0)
    tiny = np.float32(np.finfo(np.float32).tiny)
    u = np.maximum(tiny, floats * (np.float32(1.0) - tiny) + tiny)
    return -np.log(-np.log(u.astype(np.float32))).astype(np.float32)


_EXP_GUMBEL = np.exp(_gumbel_f32(42, _VOCAB).astype(np.float64)).astype(
    np.float32)

_GDN = jax.lax.GatherDimensionNumbers(
    offset_dims=(), collapsed_slice_dims=(0,), start_index_map=(0,))


def _take16(v, perm):
    return lax.gather(
        v, perm[:, None], _GDN, slice_sizes=(1,),
        mode=lax.GatherScatterMode.PROMISE_IN_BOUNDS)


def _lex_better(v, bv, gi, bi):
    return (v > bv) | ((v == bv) & (gi < bi))


def _sc_body(t_hbm, w_hbm, eg_hbm, out_hbm,
             tvec_v, rows_v, tail_v, eg_v, egt_v, bv_v, bi_v, shv, shi,
             allv_v, alli_v, outbuf_v, sem):
    cid = lax.axis_index("c")
    sid = lax.axis_index("s")
    lanes = lax.iota(jnp.int32, 16)

    start = jnp.minimum(sid * _CHUNK, _TAIL - _CHUNK)
    pltpu.sync_copy(t_hbm, tvec_v)
    t8 = pl.multiple_of(
        lax.squeeze(lax.slice(tvec_v[...], (0,), (1,)), (0,)), 8)
    r = lax.squeeze(lax.slice(tvec_v[...], (1,), (2,)), (0,))
    gather = pltpu.async_copy(
        w_hbm.at[pl.ds(t8, 8), pl.ds(start, _CHUNK)], rows_v, sem)
    pltpu.sync_copy(w_hbm.at[pl.ds(t8, 8), pl.ds(_TAIL, 32)], tail_v)
    pltpu.sync_copy(eg_hbm.at[pl.ds(start, _CHUNK)], eg_v)
    pltpu.sync_copy(eg_hbm.at[pl.ds(_TAIL, 32)], egt_v)
    gather.wait()

    def scan_body(i, carry):
        bv, bi = carry
        off = i * 16
        v = rows_v[r, pl.ds(off, 16)] * eg_v[pl.ds(off, 16)]
        gi = start + off + lanes
        better = _lex_better(v, bv, gi, bi)
        return jnp.where(better, v, bv), jnp.where(better, gi, bi)

    bv, bi = lax.fori_loop(
        0, _NSCAN, scan_body,
        (jnp.full((16,), -jnp.inf, jnp.float32),
         jnp.zeros((16,), jnp.int32)))

    for k in range(2):
        v = tail_v[r, pl.ds(k * 16, 16)] * egt_v[pl.ds(k * 16, 16)]
        gi = _TAIL + k * 16 + lanes
        better = _lex_better(v, bv, gi, bi)
        bv = jnp.where(better, v, bv)
        bi = jnp.where(better, gi, bi)

    bv_v[...] = bv
    bi_v[...] = bi
    pltpu.sync_copy(bv_v, shv.at[pl.ds(sid * 16, 16)])
    pltpu.sync_copy(bi_v, shi.at[pl.ds(sid * 16, 16)])
    plsc.subcore_barrier()
    pltpu.sync_copy(shv, allv_v)
    pltpu.sync_copy(shi, alli_v)
    gv = allv_v[pl.ds(0, 16)]
    gi = alli_v[pl.ds(0, 16)]
    for j in range(1, _NSUB):
        v = allv_v[pl.ds(j * 16, 16)]
        ii = alli_v[pl.ds(j * 16, 16)]
        better = _lex_better(v, gv, ii, gi)
        gv = jnp.where(better, v, gv)
        gi = jnp.where(better, ii, gi)

    for s in (8, 4, 2, 1):
        perm = lanes ^ s
        ov = _take16(gv, perm)
        oi = _take16(gi, perm)
        better = _lex_better(ov, gv, oi, gi)
        gv = jnp.where(better, ov, gv)
        gi = jnp.where(better, oi, gi)
    amax = gi

    out_start = cid * _HALF + jnp.minimum(sid * _OUT_CHUNK, _HALF - _OUT_CHUNK)

    def fill_body(j, carry):
        off = j * 16
        gidx = out_start + off + lanes
        outbuf_v[pl.ds(off, 16)] = jnp.where(gidx == amax, 0.0, 1000.0)
        return carry

    lax.fori_loop(0, _NOUT, fill_body, 0)
    pltpu.sync_copy(outbuf_v, out_hbm.at[pl.ds(out_start, _OUT_CHUNK)])


@functools.cache
def _sc_call():
    return pl.kernel(
        _sc_body,
        out_type=jax.ShapeDtypeStruct((_VOCAB,), jnp.float32),
        mesh=plsc.VectorSubcoreMesh(core_axis_name="c", subcore_axis_name="s"),
        scratch_types=[
            pltpu.VMEM((2,), jnp.int32),
            pltpu.VMEM((8, _CHUNK), jnp.float32),
            pltpu.VMEM((8, 32), jnp.float32),
            pltpu.VMEM((_CHUNK,), jnp.float32),
            pltpu.VMEM((32,), jnp.float32),
            pltpu.VMEM((16,), jnp.float32),
            pltpu.VMEM((16,), jnp.int32),
            pltpu.VMEM_SHARED((256,), jnp.float32),
            pltpu.VMEM_SHARED((256,), jnp.int32),
            pltpu.VMEM((256,), jnp.float32),
            pltpu.VMEM((256,), jnp.int32),
            pltpu.VMEM((_OUT_CHUNK,), jnp.float32),
            pltpu.SemaphoreType.DMA,
        ],
    )


def kernel(weights, time):
    eg = jnp.asarray(_EXP_GUMBEL)
    t = jnp.asarray(time, jnp.int32)
    taux = jnp.stack([(t // 8) * 8, t % 8])
    return _sc_call()(taux, weights, eg)


# speedup vs baseline: 2.2645x; 1.0153x over previous
import functools

import jax
import jax.numpy as jnp
import numpy as np
from jax import lax
from jax.experimental import pallas as pl
from jax.experimental.pallas import tpu as pltpu
from jax.experimental.pallas import tpu_sc as plsc

_T = 512
_VOCAB = 100000
_NSUB = 16
_CHUNK = 6272
_NSCAN = _CHUNK // 16
_HALF = _VOCAB // 2
_OUT_CHUNK = 3136
_NOUT = _OUT_CHUNK // 16


def _rotl(x, r):
    return ((x << np.uint32(r)) | (x >> np.uint32(32 - r))).astype(np.uint32)


def _threefry2x32(k0, k1, x0, x1):
    rot = ((13, 15, 26, 6), (17, 29, 16, 24))
    ks = (np.uint32(k0), np.uint32(k1),
          np.uint32(k0 ^ k1 ^ np.uint32(0x1BD11BDA)))
    x0 = (x0 + ks[0]).astype(np.uint32)
    x1 = (x1 + ks[1]).astype(np.uint32)
    for i in range(5):
        for r in rot[i % 2]:
            x0 = (x0 + x1).astype(np.uint32)
            x1 = _rotl(x1, r)
            x1 = x1 ^ x0
        x0 = (x0 + ks[(i + 1) % 3]).astype(np.uint32)
        x1 = (x1 + ks[(i + 2) % 3] + np.uint32(i + 1)).astype(np.uint32)
    return x0, x1


def _gumbel_f32(seed, n):
    k0 = np.uint32(np.uint64(seed) >> np.uint64(32))
    k1 = np.uint32(np.uint64(seed) & np.uint64(0xFFFFFFFF))
    count = np.arange(n, dtype=np.uint32)
    o0, o1 = _threefry2x32(k0, k1, np.zeros(n, np.uint32), count)
    bits = o0 ^ o1
    float_bits = (bits >> np.uint32(9)) | np.uint32(0x3F800000)
    floats = float_bits.view(np.float32) - np.float32(1.0)
    tiny = np.float32(np.finfo(np.float32).tiny)
    u = np.maximum(tiny, floats * (np.float32(1.0) - tiny) + tiny)
    return -np.log(-np.log(u.astype(np.float32))).astype(np.float32)


_EXP_GUMBEL = np.exp(_gumbel_f32(42, _VOCAB).astype(np.float64)).astype(
    np.float32)

_GDN = jax.lax.GatherDimensionNumbers(
    offset_dims=(), collapsed_slice_dims=(0,), start_index_map=(0,))


def _take16(v, perm):
    return lax.gather(
        v, perm[:, None], _GDN, slice_sizes=(1,),
        mode=lax.GatherScatterMode.PROMISE_IN_BOUNDS)


def _lex_better(v, bv, gi, bi):
    return (v > bv) | ((v == bv) & (gi < bi))


def _tc_gather_body(t_ref, w_ref, out_ref):
    out_ref[...] = w_ref[t_ref[0] % 8]


@functools.cache
def _tc_gather():
    return pl.pallas_call(
        _tc_gather_body,
        grid_spec=pltpu.PrefetchScalarGridSpec(
            num_scalar_prefetch=1,
            grid=(1,),
            in_specs=[pl.BlockSpec((8, _VOCAB), lambda i, t: (t[0] // 8, 0))],
            out_specs=pl.BlockSpec((_VOCAB,), lambda i, t: (0,)),
        ),
        out_shape=jax.ShapeDtypeStruct((_VOCAB,), jnp.float32),
    )


def _sc_body(w_hbm, eg_hbm, out_hbm,
             rows_v, eg_v, bv_v, bi_v, shv, shi, allv_v, alli_v,
             outbuf_v, sem):
    cid = lax.axis_index("c")
    sid = lax.axis_index("s")
    lanes = lax.iota(jnp.int32, 16)

    start = jnp.minimum(sid * _CHUNK, _VOCAB - _CHUNK)
    gather = pltpu.async_copy(w_hbm.at[pl.ds(start, _CHUNK)], rows_v, sem)
    pltpu.sync_copy(eg_hbm.at[pl.ds(start, _CHUNK)], eg_v)
    gather.wait()

    def scan_body(i, carry):
        bv, bi = carry
        off = i * 16
        v = rows_v[pl.ds(off, 16)] * eg_v[pl.ds(off, 16)]
        gi = start + off + lanes
        better = _lex_better(v, bv, gi, bi)
        return jnp.where(better, v, bv), jnp.where(better, gi, bi)

    bv, bi = lax.fori_loop(
        0, _NSCAN, scan_body,
        (jnp.full((16,), -jnp.inf, jnp.float32),
         jnp.zeros((16,), jnp.int32)))

    bv_v[...] = bv
    bi_v[...] = bi
    pltpu.sync_copy(bv_v, shv.at[pl.ds(sid * 16, 16)])
    pltpu.sync_copy(bi_v, shi.at[pl.ds(sid * 16, 16)])
    plsc.subcore_barrier()
    pltpu.sync_copy(shv, allv_v)
    pltpu.sync_copy(shi, alli_v)
    gv = allv_v[pl.ds(0, 16)]
    gi = alli_v[pl.ds(0, 16)]
    for j in range(1, _NSUB):
        v = allv_v[pl.ds(j * 16, 16)]
        ii = alli_v[pl.ds(j * 16, 16)]
        better = _lex_better(v, gv, ii, gi)
        gv = jnp.where(better, v, gv)
        gi = jnp.where(better, ii, gi)

    for s in (8, 4, 2, 1):
        perm = lanes ^ s
        ov = _take16(gv, perm)
        oi = _take16(gi, perm)
        better = _lex_better(ov, gv, oi, gi)
        gv = jnp.where(better, ov, gv)
        gi = jnp.where(better, oi, gi)
    amax = gi

    out_start = cid * _HALF + jnp.minimum(sid * _OUT_CHUNK, _HALF - _OUT_CHUNK)

    def fill_body(j, carry):
        off = j * 16
        gidx = out_start + off + lanes
        outbuf_v[pl.ds(off, 16)] = jnp.where(gidx == amax, 0.0, 1000.0)
        return carry

    lax.fori_loop(0, _NOUT, fill_body, 0)
    pltpu.sync_copy(outbuf_v, out_hbm.at[pl.ds(out_start, _OUT_CHUNK)])


@functools.cache
def _sc_call():
    return pl.kernel(
        _sc_body,
        out_type=jax.ShapeDtypeStruct((_VOCAB,), jnp.float32),
        mesh=plsc.VectorSubcoreMesh(core_axis_name="c", subcore_axis_name="s"),
        scratch_types=[
            pltpu.VMEM((_CHUNK,), jnp.float32),
            pltpu.VMEM((_CHUNK,), jnp.float32),
            pltpu.VMEM((16,), jnp.float32),
            pltpu.VMEM((16,), jnp.int32),
            pltpu.VMEM_SHARED((256,), jnp.float32),
            pltpu.VMEM_SHARED((256,), jnp.int32),
            pltpu.VMEM((256,), jnp.float32),
            pltpu.VMEM((256,), jnp.int32),
            pltpu.VMEM((_OUT_CHUNK,), jnp.float32),
            pltpu.SemaphoreType.DMA,
        ],
    )


def kernel(weights, time):
    eg = jnp.asarray(_EXP_GUMBEL)
    t = jnp.asarray(time, jnp.int32)
    row = _tc_gather()(t[None], weights)
    return _sc_call()(row, eg)


# speedup vs baseline: 5.6883x; 2.5119x over previous
import functools

import jax
import jax.numpy as jnp
import numpy as np
from jax import lax
from jax.experimental import pallas as pl
from jax.experimental.pallas import tpu as pltpu
from jax.experimental.pallas import tpu_sc as plsc

_T = 512
_VOCAB = 100000
_NSUB = 16
_CHUNK = 6272
_NSCAN = _CHUNK // 16
_HALF = _VOCAB // 2
_OUT_CHUNK = 3136
_NOUT = _OUT_CHUNK // 16


def _rotl(x, r):
    return ((x << np.uint32(r)) | (x >> np.uint32(32 - r))).astype(np.uint32)


def _threefry2x32(k0, k1, x0, x1):
    rot = ((13, 15, 26, 6), (17, 29, 16, 24))
    ks = (np.uint32(k0), np.uint32(k1),
          np.uint32(k0 ^ k1 ^ np.uint32(0x1BD11BDA)))
    x0 = (x0 + ks[0]).astype(np.uint32)
    x1 = (x1 + ks[1]).astype(np.uint32)
    for i in range(5):
        for r in rot[i % 2]:
            x0 = (x0 + x1).astype(np.uint32)
            x1 = _rotl(x1, r)
            x1 = x1 ^ x0
        x0 = (x0 + ks[(i + 1) % 3]).astype(np.uint32)
        x1 = (x1 + ks[(i + 2) % 3] + np.uint32(i + 1)).astype(np.uint32)
    return x0, x1


def _gumbel_f32(seed, n):
    k0 = np.uint32(np.uint64(seed) >> np.uint64(32))
    k1 = np.uint32(np.uint64(seed) & np.uint64(0xFFFFFFFF))
    count = np.arange(n, dtype=np.uint32)
    o0, o1 = _threefry2x32(k0, k1, np.zeros(n, np.uint32), count)
    bits = o0 ^ o1
    float_bits = (bits >> np.uint32(9)) | np.uint32(0x3F800000)
    floats = float_bits.view(np.float32) - np.float32(1.0)
    tiny = np.float32(np.finfo(np.float32).tiny)
    u = np.maximum(tiny, floats * (np.float32(1.0) - tiny) + tiny)
    return -np.log(-np.log(u.astype(np.float32))).astype(np.float32)


_EXP_GUMBEL = np.exp(_gumbel_f32(42, _VOCAB).astype(np.float64)).astype(
    np.float32)

_GDN = jax.lax.GatherDimensionNumbers(
    offset_dims=(), collapsed_slice_dims=(0,), start_index_map=(0,))


def _take16(v, perm):
    return lax.gather(
        v, perm[:, None], _GDN, slice_sizes=(1,),
        mode=lax.GatherScatterMode.PROMISE_IN_BOUNDS)


def _lex_better(v, bv, gi, bi):
    return (v > bv) | ((v == bv) & (gi < bi))


_VB = 5000


def _tc_gather_body(t_ref, w_ref, out_ref):
    c = t_ref[0] % 128
    lane = lax.broadcasted_iota(jnp.int32, (_VB, 128), 1)
    w = w_ref[...]
    out_ref[...] = jnp.sum(jnp.where(lane == c, w, 0.0), axis=1,
                           keepdims=True)


@functools.cache
def _tc_gather():
    return pl.pallas_call(
        _tc_gather_body,
        grid_spec=pltpu.PrefetchScalarGridSpec(
            num_scalar_prefetch=1,
            grid=(_VOCAB // _VB,),
            in_specs=[pl.BlockSpec((_VB, 128), lambda i, t: (i, t[0] // 128))],
            out_specs=pl.BlockSpec((_VB, 1), lambda i, t: (i, 0)),
        ),
        out_shape=jax.ShapeDtypeStruct((_VOCAB, 1), jnp.float32),
    )


def _sc_body(w_hbm, eg_hbm, out_hbm,
             rows_v, eg_v, bv_v, bi_v, shv, shi, allv_v, alli_v,
             outbuf_v, sem):
    cid = lax.axis_index("c")
    sid = lax.axis_index("s")
    lanes = lax.iota(jnp.int32, 16)

    start = jnp.minimum(sid * _CHUNK, _VOCAB - _CHUNK)
    gather = pltpu.async_copy(w_hbm.at[pl.ds(start, _CHUNK)], rows_v, sem)
    pltpu.sync_copy(eg_hbm.at[pl.ds(start, _CHUNK)], eg_v)
    gather.wait()

    def scan_body(i, carry):
        bv, bi = carry
        off = i * 16
        v = rows_v[pl.ds(off, 16)] * eg_v[pl.ds(off, 16)]
        gi = start + off + lanes
        better = _lex_better(v, bv, gi, bi)
        return jnp.where(better, v, bv), jnp.where(better, gi, bi)

    bv, bi = lax.fori_loop(
        0, _NSCAN, scan_body,
        (jnp.full((16,), -jnp.inf, jnp.float32),
         jnp.zeros((16,), jnp.int32)))

    bv_v[...] = bv
    bi_v[...] = bi
    pltpu.sync_copy(bv_v, shv.at[pl.ds(sid * 16, 16)])
    pltpu.sync_copy(bi_v, shi.at[pl.ds(sid * 16, 16)])
    plsc.subcore_barrier()
    pltpu.sync_copy(shv, allv_v)
    pltpu.sync_copy(shi, alli_v)
    gv = allv_v[pl.ds(0, 16)]
    gi = alli_v[pl.ds(0, 16)]
    for j in range(1, _NSUB):
        v = allv_v[pl.ds(j * 16, 16)]
        ii = alli_v[pl.ds(j * 16, 16)]
        better = _lex_better(v, gv, ii, gi)
        gv = jnp.where(better, v, gv)
        gi = jnp.where(better, ii, gi)

    for s in (8, 4, 2, 1):
        perm = lanes ^ s
        ov = _take16(gv, perm)
        oi = _take16(gi, perm)
        better = _lex_better(ov, gv, oi, gi)
        gv = jnp.where(better, ov, gv)
        gi = jnp.where(better, oi, gi)
    amax = gi

    out_start = cid * _HALF + jnp.minimum(sid * _OUT_CHUNK, _HALF - _OUT_CHUNK)

    def fill_body(j, carry):
        off = j * 16
        gidx = out_start + off + lanes
        outbuf_v[pl.ds(off, 16)] = jnp.where(gidx == amax, 0.0, 1000.0)
        return carry

    lax.fori_loop(0, _NOUT, fill_body, 0)
    pltpu.sync_copy(outbuf_v, out_hbm.at[pl.ds(out_start, _OUT_CHUNK)])


@functools.cache
def _sc_call():
    return pl.kernel(
        _sc_body,
        out_type=jax.ShapeDtypeStruct((_VOCAB,), jnp.float32),
        mesh=plsc.VectorSubcoreMesh(core_axis_name="c", subcore_axis_name="s"),
        scratch_types=[
            pltpu.VMEM((_CHUNK,), jnp.float32),
            pltpu.VMEM((_CHUNK,), jnp.float32),
            pltpu.VMEM((16,), jnp.float32),
            pltpu.VMEM((16,), jnp.int32),
            pltpu.VMEM_SHARED((256,), jnp.float32),
            pltpu.VMEM_SHARED((256,), jnp.int32),
            pltpu.VMEM((256,), jnp.float32),
            pltpu.VMEM((256,), jnp.int32),
            pltpu.VMEM((_OUT_CHUNK,), jnp.float32),
            pltpu.SemaphoreType.DMA,
        ],
    )


def kernel(weights, time):
    eg = jnp.asarray(_EXP_GUMBEL)
    t = jnp.asarray(time, jnp.int32)
    row = _tc_gather()(t[None], weights.T).reshape(_VOCAB)
    return _sc_call()(row, eg)
